# baseline (speedup 1.0000x reference)
import jax
import jax.numpy as jnp
from jax.experimental import pallas as pl

N = 2048
HEADS = 4


def _gat_conv(x, edge_index, W, att_src, att_dst, bias, heads, concat):
    n = x.shape[0]
    x_l = (x @ W).reshape(n, heads, -1)
    loops = jnp.arange(n, dtype=edge_index.dtype)
    src = jnp.concatenate([edge_index[0], loops])
    dst = jnp.concatenate([edge_index[1], loops])
    a_src = (x_l * att_src).sum(-1)
    a_dst = (x_l * att_dst).sum(-1)
    alpha = a_src[src] + a_dst[dst]
    alpha = jax.nn.leaky_relu(alpha, 0.2)
    m = jax.ops.segment_max(alpha, dst, num_segments=n)
    m = jnp.where(jnp.isfinite(m), m, 0.0)
    e = jnp.exp(alpha - m[dst])
    denom = jax.ops.segment_sum(e, dst, num_segments=n)
    a = e / (denom[dst] + 1e-16)
    msg = x_l[src] * a[:, :, None]
    out = jax.ops.segment_sum(msg, dst, num_segments=n)
    if concat:
        out = out.reshape(n, heads * out.shape[-1])
    else:
        out = out.mean(axis=1)
    return out + bias


def _decoder_body(zm_ref, w_ref, b_ref, out_ref):
    acc = b_ref[...]
    for h in range(16):
        acc = acc + zm_ref[0, h] * w_ref[h]
    out_ref[...] = jax.nn.sigmoid(acc)


def _decode(zm, dec_W, dec_b):
    w3 = dec_W.reshape(16, N, N)
    b2 = dec_b.reshape(N, N)
    zm2 = zm.reshape(1, 16)
    BR = 32
    return pl.pallas_call(
        _decoder_body,
        grid=(N // BR,),
        in_specs=[
            pl.BlockSpec((1, 16), lambda i: (0, 0)),
            pl.BlockSpec((16, BR, N), lambda i: (0, i, 0)),
            pl.BlockSpec((BR, N), lambda i: (i, 0)),
        ],
        out_specs=pl.BlockSpec((BR, N), lambda i: (i, 0)),
        out_shape=jax.ShapeDtypeStruct((N, N), jnp.float32),
    )(zm2, w3, b2)


def kernel(edge_index, x, W1, att_src1, att_dst1, b1, W2, att_src2, att_dst2,
           b2, mu_W, mu_b, lv_W, lv_b, dec_W, dec_b):
    hidden = jax.nn.relu(_gat_conv(x, edge_index, W1, att_src1, att_dst1, b1, HEADS, True))
    embedding = _gat_conv(hidden, edge_index, W2, att_src2, att_dst2, b2, 1, False)
    mu = embedding @ mu_W + mu_b
    log_var = embedding @ lv_W + lv_b
    std = jnp.exp(0.5 * log_var)
    eps = jax.random.normal(jax.random.key(42), std.shape, dtype=std.dtype)
    z = mu + eps * std
    zm = z.mean(axis=0)
    return _decode(zm, dec_W, dec_b)
